# baseline (speedup 1.0000x reference)
import functools

import jax
import jax.numpy as jnp
from jax import lax
from jax.experimental import pallas as pl
from jax.experimental.pallas import tpu as pltpu
from jax.experimental.pallas import tpu_sc as plsc

N_ROWS = 64
NNZ = 4
N_COLS = 256
BATCH = 4096

NUM_WORKERS = 32
ROWS_PER_WORKER = BATCH // NUM_WORKERS
LANES = 16
N_OG = N_ROWS // LANES


def _sc_call(x, idx16, w16):
    mesh = plsc.VectorSubcoreMesh(core_axis_name="c", subcore_axis_name="s")

    @functools.partial(
        pl.kernel,
        mesh=mesh,
        out_type=jax.ShapeDtypeStruct((BATCH, N_ROWS), jnp.float32),
        compiler_params=pltpu.CompilerParams(
            use_tc_tiling_on_sc=False, needs_layout_passes=False),
        scratch_types=[
            pltpu.VMEM((ROWS_PER_WORKER, N_COLS), jnp.float32),
            pltpu.VMEM((ROWS_PER_WORKER, N_ROWS), jnp.float32),
            pltpu.VMEM((N_OG * NNZ, LANES), jnp.int32),
            pltpu.VMEM((N_OG * NNZ, LANES), jnp.float32),
        ],
    )
    def sc_kernel(x_hbm, idx_hbm, w_hbm, out_hbm, x_v, y_v, idx_v, w_v):
        wid = lax.axis_index("s") * 2 + lax.axis_index("c")
        b0 = wid * ROWS_PER_WORKER
        pltpu.sync_copy(idx_hbm, idx_v)
        pltpu.sync_copy(w_hbm, w_v)
        pltpu.sync_copy(x_hbm.at[pl.ds(b0, ROWS_PER_WORKER), :], x_v)

        def body(b, _):
            bvec = jnp.full((LANES,), b, jnp.int32)
            for og in range(N_OG):
                acc = None
                for r in range(NNZ):
                    g = plsc.load_gather(x_v, [bvec, idx_v[og * NNZ + r]])
                    t = g * w_v[og * NNZ + r]
                    acc = t if acc is None else acc + t
                y_v[b, pl.ds(og * LANES, LANES)] = acc
            return _

        lax.fori_loop(0, ROWS_PER_WORKER, body, None)
        pltpu.sync_copy(y_v, out_hbm.at[pl.ds(b0, ROWS_PER_WORKER), :])

    return sc_kernel(x, idx16, w16)


def kernel(inputs, sparse_kernel, ij):
    cols = ij[:, 1].astype(jnp.int32)
    w = sparse_kernel[:, 0].astype(jnp.float32)
    idx16 = cols.reshape(N_OG, LANES, NNZ).transpose(0, 2, 1).reshape(
        N_OG * NNZ, LANES)
    w16 = w.reshape(N_OG, LANES, NNZ).transpose(0, 2, 1).reshape(
        N_OG * NNZ, LANES)
    return _sc_call(inputs, idx16, w16)


# speedup vs baseline: 1.1050x; 1.1050x over previous
import functools

import jax
import jax.numpy as jnp
from jax import lax
from jax.experimental import pallas as pl
from jax.experimental.pallas import tpu as pltpu
from jax.experimental.pallas import tpu_sc as plsc

N_ROWS = 64
NNZ = 4
N_COLS = 256
BATCH = 4096

NUM_WORKERS = 32
ROWS_PER_WORKER = BATCH // NUM_WORKERS
LANES = 16
N_OG = N_ROWS // LANES
N_VEC = N_OG * NNZ


def _sc_call(x_flat, idx16, w16):
    mesh = plsc.VectorSubcoreMesh(core_axis_name="c", subcore_axis_name="s")

    @functools.partial(
        pl.kernel,
        mesh=mesh,
        out_type=jax.ShapeDtypeStruct((BATCH, N_ROWS), jnp.float32),
        compiler_params=pltpu.CompilerParams(
            use_tc_tiling_on_sc=False, needs_layout_passes=False),
        scratch_types=[
            pltpu.VMEM((ROWS_PER_WORKER * N_COLS,), jnp.float32),
            pltpu.VMEM((ROWS_PER_WORKER, N_ROWS), jnp.float32),
            pltpu.VMEM((N_VEC, LANES), jnp.int32),
            pltpu.VMEM((N_VEC, LANES), jnp.float32),
        ],
    )
    def sc_kernel(x_hbm, idx_hbm, w_hbm, out_hbm, x_v, y_v, idx_v, w_v):
        wid = lax.axis_index("s") * 2 + lax.axis_index("c")
        b0 = wid * ROWS_PER_WORKER
        pltpu.sync_copy(idx_hbm, idx_v)
        pltpu.sync_copy(w_hbm, w_v)
        pltpu.sync_copy(
            x_hbm.at[pl.ds(b0 * N_COLS, ROWS_PER_WORKER * N_COLS)], x_v)

        ws = [w_v[j] for j in range(N_VEC)]
        idx0 = tuple(idx_v[j] for j in range(N_VEC))
        step = jnp.full((LANES,), N_COLS, jnp.int32)

        def body(b, idxs):
            for og in range(N_OG):
                acc = None
                for r in range(NNZ):
                    j = og * NNZ + r
                    g = plsc.load_gather(x_v, [idxs[j]])
                    t = g * ws[j]
                    acc = t if acc is None else acc + t
                y_v[b, pl.ds(og * LANES, LANES)] = acc
            return tuple(ix + step for ix in idxs)

        lax.fori_loop(0, ROWS_PER_WORKER, body, idx0)
        pltpu.sync_copy(y_v, out_hbm.at[pl.ds(b0, ROWS_PER_WORKER), :])

    return sc_kernel(x_flat, idx16, w16)


def kernel(inputs, sparse_kernel, ij):
    cols = ij[:, 1].astype(jnp.int32)
    w = sparse_kernel[:, 0].astype(jnp.float32)
    idx16 = cols.reshape(N_OG, LANES, NNZ).transpose(0, 2, 1).reshape(
        N_VEC, LANES)
    w16 = w.reshape(N_OG, LANES, NNZ).transpose(0, 2, 1).reshape(
        N_VEC, LANES)
    return _sc_call(inputs.reshape(-1), idx16, w16)


# speedup vs baseline: 1.2226x; 1.1064x over previous
import functools

import jax
import jax.numpy as jnp
from jax import lax
from jax.experimental import pallas as pl
from jax.experimental.pallas import tpu as pltpu
from jax.experimental.pallas import tpu_sc as plsc

N_ROWS = 64
NNZ = 4
N_COLS = 256
BATCH = 4096

NUM_WORKERS = 32
ROWS_PER_WORKER = BATCH // NUM_WORKERS
LANES = 16
N_OG = N_ROWS // LANES
N_VEC = N_OG * NNZ
N_CHUNKS = 4
CHUNK = ROWS_PER_WORKER // N_CHUNKS


def _sc_call(x_flat, idx16, w16):
    mesh = plsc.VectorSubcoreMesh(core_axis_name="c", subcore_axis_name="s")

    @functools.partial(
        pl.kernel,
        mesh=mesh,
        out_type=jax.ShapeDtypeStruct((BATCH, N_ROWS), jnp.float32),
        compiler_params=pltpu.CompilerParams(
            use_tc_tiling_on_sc=False, needs_layout_passes=False),
        scratch_types=[
            pltpu.VMEM((2, CHUNK * N_COLS), jnp.float32),
            pltpu.VMEM((2, CHUNK, N_ROWS), jnp.float32),
            pltpu.VMEM((N_VEC, LANES), jnp.int32),
            pltpu.VMEM((N_VEC, LANES), jnp.float32),
            pltpu.SemaphoreType.DMA,
            pltpu.SemaphoreType.DMA,
            pltpu.SemaphoreType.DMA,
            pltpu.SemaphoreType.DMA,
        ],
    )
    def sc_kernel(x_hbm, idx_hbm, w_hbm, out_hbm, x_v, y_v, idx_v, w_v,
                  in_sem0, in_sem1, out_sem0, out_sem1):
        wid = lax.axis_index("s") * 2 + lax.axis_index("c")
        b0 = wid * ROWS_PER_WORKER
        in_sems = (in_sem0, in_sem1)
        out_sems = (out_sem0, out_sem1)

        def start_in(c):
            return pltpu.async_copy(
                x_hbm.at[pl.ds((b0 + c * CHUNK) * N_COLS, CHUNK * N_COLS)],
                x_v.at[c % 2], in_sems[c % 2])

        cp0 = start_in(0)
        cp1 = start_in(1)
        in_cps = [cp0, cp1]

        pltpu.sync_copy(idx_hbm, idx_v)
        pltpu.sync_copy(w_hbm, w_v)
        ws = [w_v[j] for j in range(N_VEC)]
        idxs = [idx_v[j] for j in range(N_VEC)]

        out_cps = [None, None]
        for c in range(N_CHUNKS):
            buf = c % 2
            in_cps[buf].wait()
            if out_cps[buf] is not None:
                out_cps[buf].wait()
            xb = x_v.at[buf]
            yb = y_v.at[buf]

            @plsc.parallel_loop(0, CHUNK, unroll=2)
            def _loop(b):
                row = xb.at[pl.ds(b * N_COLS, N_COLS)]
                for og in range(N_OG):
                    j = og * NNZ
                    t0 = plsc.load_gather(row, [idxs[j]]) * ws[j]
                    t1 = plsc.load_gather(row, [idxs[j + 1]]) * ws[j + 1]
                    t2 = plsc.load_gather(row, [idxs[j + 2]]) * ws[j + 2]
                    t3 = plsc.load_gather(row, [idxs[j + 3]]) * ws[j + 3]
                    yb[b, pl.ds(og * LANES, LANES)] = (t0 + t1) + (t2 + t3)
            out_cps[buf] = pltpu.async_copy(
                yb, out_hbm.at[pl.ds(b0 + c * CHUNK, CHUNK), :],
                out_sems[buf])
            if c + 2 < N_CHUNKS:
                in_cps[buf] = start_in(c + 2)
        out_cps[0].wait()
        out_cps[1].wait()

    return sc_kernel(x_flat, idx16, w16)


def kernel(inputs, sparse_kernel, ij):
    cols = ij[:, 1].astype(jnp.int32)
    w = sparse_kernel[:, 0].astype(jnp.float32)
    idx16 = cols.reshape(N_OG, LANES, NNZ).transpose(0, 2, 1).reshape(
        N_VEC, LANES)
    w16 = w.reshape(N_OG, LANES, NNZ).transpose(0, 2, 1).reshape(
        N_VEC, LANES)
    return _sc_call(inputs.reshape(-1), idx16, w16)


# speedup vs baseline: 1.3241x; 1.0830x over previous
import functools

import jax
import jax.numpy as jnp
from jax import lax
from jax.experimental import pallas as pl
from jax.experimental.pallas import tpu as pltpu
from jax.experimental.pallas import tpu_sc as plsc

N_ROWS = 64
NNZ = 4
N_COLS = 256
BATCH = 4096

NUM_WORKERS = 32
ROWS_PER_WORKER = BATCH // NUM_WORKERS
LANES = 16
N_OG = N_ROWS // LANES
N_VEC = N_OG * NNZ
N_CHUNKS = 4
CHUNK = ROWS_PER_WORKER // N_CHUNKS


def _sc_call(x, ij_flat, w_flat):
    mesh = plsc.VectorSubcoreMesh(core_axis_name="c", subcore_axis_name="s")

    @functools.partial(
        pl.kernel,
        mesh=mesh,
        out_type=jax.ShapeDtypeStruct((BATCH, N_ROWS), jnp.float32),
        compiler_params=pltpu.CompilerParams(
            use_tc_tiling_on_sc=False, needs_layout_passes=False),
        scratch_types=[
            pltpu.VMEM((2, CHUNK, N_COLS), jnp.float32),
            pltpu.VMEM((2, CHUNK, N_ROWS), jnp.float32),
            pltpu.VMEM((2 * N_VEC * LANES,), jnp.int32),
            pltpu.VMEM((N_VEC * LANES,), jnp.float32),
            pltpu.SemaphoreType.DMA,
            pltpu.SemaphoreType.DMA,
            pltpu.SemaphoreType.DMA,
            pltpu.SemaphoreType.DMA,
        ],
    )
    def sc_kernel(x_hbm, ij_hbm, w_hbm, out_hbm, x_v, y_v, ij_v, w_v,
                  in_sem0, in_sem1, out_sem0, out_sem1):
        wid = lax.axis_index("s") * 2 + lax.axis_index("c")
        b0 = wid * ROWS_PER_WORKER
        in_sems = (in_sem0, in_sem1)
        out_sems = (out_sem0, out_sem1)

        def start_in(c):
            return pltpu.async_copy(
                x_hbm.at[pl.ds(b0 + c * CHUNK, CHUNK), :],
                x_v.at[c % 2], in_sems[c % 2])

        in_cps = [start_in(0), start_in(1)]

        pltpu.sync_copy(ij_hbm, ij_v)
        pltpu.sync_copy(w_hbm, w_v)
        lane = lax.iota(jnp.int32, LANES)
        ws = []
        idxs = []
        for og in range(N_OG):
            for r in range(NNZ):
                k = lane * NNZ + (og * LANES * NNZ + r)
                ws.append(plsc.load_gather(w_v, [k]))
                idxs.append(plsc.load_gather(ij_v, [k * 2 + 1]))

        out_cps = [None, None]
        for c in range(N_CHUNKS):
            buf = c % 2
            in_cps[buf].wait()
            if out_cps[buf] is not None:
                out_cps[buf].wait()
            xb = x_v.at[buf]
            yb = y_v.at[buf]

            @plsc.parallel_loop(0, CHUNK, unroll=2)
            def _loop(b):
                row = xb.at[b]
                for og in range(N_OG):
                    j = og * NNZ
                    t0 = plsc.load_gather(row, [idxs[j]]) * ws[j]
                    t1 = plsc.load_gather(row, [idxs[j + 1]]) * ws[j + 1]
                    t2 = plsc.load_gather(row, [idxs[j + 2]]) * ws[j + 2]
                    t3 = plsc.load_gather(row, [idxs[j + 3]]) * ws[j + 3]
                    yb[b, pl.ds(og * LANES, LANES)] = (t0 + t1) + (t2 + t3)

            out_cps[buf] = pltpu.async_copy(
                yb, out_hbm.at[pl.ds(b0 + c * CHUNK, CHUNK), :],
                out_sems[buf])
            if c + 2 < N_CHUNKS:
                in_cps[buf] = start_in(c + 2)
        out_cps[0].wait()
        out_cps[1].wait()

    return sc_kernel(x, ij_flat, w_flat)


def kernel(inputs, sparse_kernel, ij):
    return _sc_call(
        inputs,
        ij.astype(jnp.int32).reshape(-1),
        sparse_kernel.astype(jnp.float32).reshape(-1),
    )
